# baseline (speedup 1.0000x reference)
import functools

import jax
import jax.numpy as jnp
from jax import lax
from jax.experimental import pallas as pl
from jax.experimental.pallas import tpu as pltpu
from jax.experimental.pallas import tpu_sc as plsc

VOCAB = 1000
B, T = 1024, 50
NC, NS = 2, 16
NW = NC * NS
DPW = 32
DLAST = VOCAB - (NW - 1) * DPW
NVEC = B // 16

_mesh = plsc.VectorSubcoreMesh(core_axis_name="c", subcore_axis_name="s")


@functools.partial(
    pl.kernel,
    mesh=_mesh,
    out_type=jax.ShapeDtypeStruct((T, VOCAB, B), jnp.float32),
    scratch_types=[
        pltpu.VMEM((DPW * 1024,), jnp.float32),
        [pltpu.VMEM((B,), jnp.int32) for _ in range(2)],
        [pltpu.VMEM((DPW, B), jnp.float32) for _ in range(2)],
        [pltpu.SemaphoreType.DMA for _ in range(2)],
        [pltpu.SemaphoreType.DMA for _ in range(2)],
    ],
    compiler_params=pltpu.CompilerParams(needs_layout_passes=False),
)
def _gather_kernel(tableT, idxT, out_t, tblv, idxb, obuf, isem, wsem):
    sid = lax.axis_index("s")
    wid = sid * NC + lax.axis_index("c")
    d0 = wid * DPW
    last = wid == NW - 1

    pltpu.sync_copy(tableT.at[pl.ds(d0 * 1024, DPW * 1024)], tblv)
    pltpu.async_copy(idxT.at[0], idxb[0], isem[0])

    def wait_write(p):
        @pl.when(jnp.logical_not(last))
        def _():
            pltpu.make_async_copy(
                obuf[p], out_t.at[0, pl.ds(0, DPW)], wsem[p]
            ).wait()

        @pl.when(last)
        def _():
            pltpu.make_async_copy(
                obuf[p].at[pl.ds(0, DLAST)],
                out_t.at[0, pl.ds(0, DLAST)],
                wsem[p],
            ).wait()

    def step(i, carry):
        for p in range(2):
            t = i * 2 + p
            pltpu.make_async_copy(idxT.at[0], idxb[p], isem[p]).wait()

            @pl.when(t + 1 < T)
            def _():
                pltpu.async_copy(idxT.at[t + 1], idxb[1 - p], isem[1 - p])

            @pl.when(t >= 2)
            def _():
                wait_write(p)

            @plsc.parallel_loop(0, NVEC, unroll=4)
            def jloop(j):
                idx16 = idxb[p][pl.ds(16 * j, 16)]
                for dl in range(DPW):
                    vals = plsc.load_gather(tblv, [idx16 + (dl * 1024)])
                    obuf[p][dl, pl.ds(16 * j, 16)] = vals

            @pl.when(jnp.logical_not(last))
            def _():
                pltpu.async_copy(obuf[p], out_t.at[t, pl.ds(d0, DPW)], wsem[p])

            @pl.when(last)
            def _():
                pltpu.async_copy(
                    obuf[p].at[pl.ds(0, DLAST)],
                    out_t.at[t, pl.ds(d0, DLAST)],
                    wsem[p],
                )
        return carry

    lax.fori_loop(0, T // 2, step, 0)
    wait_write(0)
    wait_write(1)


def kernel(idx, table):
    tableT = jnp.pad(table.T, ((0, NW * DPW - VOCAB), (0, 24))).reshape(-1)
    idxT = idx.T.astype(jnp.int32)
    out_t = _gather_kernel(tableT, idxT)
    return jnp.transpose(out_t, (2, 0, 1))


# speedup vs baseline: 1.2968x; 1.2968x over previous
import functools

import jax
import jax.numpy as jnp
from jax import lax
from jax.experimental import pallas as pl
from jax.experimental.pallas import tpu as pltpu
from jax.experimental.pallas import tpu_sc as plsc

VOCAB = 1000
B, T = 1024, 50
NC, NS = 2, 16
NW = NC * NS
DPW = 32
DLAST = VOCAB - (NW - 1) * DPW
NVEC = B // 16

_mesh = plsc.VectorSubcoreMesh(core_axis_name="c", subcore_axis_name="s")


@functools.partial(
    pl.kernel,
    mesh=_mesh,
    out_type=jax.ShapeDtypeStruct((T, VOCAB, B), jnp.float32),
    scratch_types=[
        pltpu.VMEM((DPW * 1024,), jnp.float32),
        [pltpu.VMEM((B,), jnp.int32) for _ in range(2)],
        [pltpu.VMEM((DPW, B), jnp.float32) for _ in range(2)],
        [pltpu.SemaphoreType.DMA for _ in range(2)],
        [pltpu.SemaphoreType.DMA for _ in range(2)],
    ],
    compiler_params=pltpu.CompilerParams(needs_layout_passes=False),
)
def _gather_kernel(tableT, idxT, out_t, tblv, idxb, obuf, isem, wsem):
    sid = lax.axis_index("s")
    wid = sid * NC + lax.axis_index("c")
    d0 = wid * DPW
    last = wid == NW - 1

    pltpu.sync_copy(tableT.at[pl.ds(d0 * 1024, DPW * 1024)], tblv)
    pltpu.async_copy(idxT.at[0], idxb[0], isem[0])

    def wait_write(p):
        @pl.when(jnp.logical_not(last))
        def _():
            pltpu.make_async_copy(
                obuf[p], out_t.at[0, pl.ds(0, DPW)], wsem[p]
            ).wait()

        @pl.when(last)
        def _():
            pltpu.make_async_copy(
                obuf[p].at[pl.ds(0, DLAST)],
                out_t.at[0, pl.ds(0, DLAST)],
                wsem[p],
            ).wait()

    def step(i, carry):
        for p in range(2):
            t = i * 2 + p
            pltpu.make_async_copy(idxT.at[0], idxb[p], isem[p]).wait()

            @pl.when(t + 1 < T)
            def _():
                pltpu.async_copy(idxT.at[t + 1], idxb[1 - p], isem[1 - p])

            @pl.when(t >= 2)
            def _():
                wait_write(p)

            @plsc.parallel_loop(0, NVEC, unroll=2)
            def jloop(j):
                idx16 = idxb[p][pl.ds(16 * j, 16)]
                for dl in range(DPW):
                    vals = plsc.load_gather(tblv, [idx16 + (dl * 1024)])
                    obuf[p][dl, pl.ds(16 * j, 16)] = vals

            @pl.when(jnp.logical_not(last))
            def _():
                pltpu.async_copy(obuf[p], out_t.at[t, pl.ds(d0, DPW)], wsem[p])

            @pl.when(last)
            def _():
                pltpu.async_copy(
                    obuf[p].at[pl.ds(0, DLAST)],
                    out_t.at[t, pl.ds(d0, DLAST)],
                    wsem[p],
                )
        return carry

    lax.fori_loop(0, T // 2, step, 0)
    wait_write(0)
    wait_write(1)


def kernel(idx, table):
    tableT = jnp.pad(table.T, ((0, NW * DPW - VOCAB), (0, 24))).reshape(-1)
    idxT = idx.T.astype(jnp.int32)
    out_t = _gather_kernel(tableT, idxT)
    return jnp.transpose(out_t, (2, 0, 1))
